# baseline (speedup 1.0000x reference)
import functools
import numpy as np
import jax
import jax.numpy as jnp
from jax import lax
from jax.experimental import pallas as pl
from jax.experimental.pallas import tpu as pltpu
from jax.experimental.pallas import tpu_sc as plsc

NUM_ACT = 3
IHT = 65536
T = 32
B = 2048
LANES = 16
NC = 2
NW = 32
BPW = B // NW
CH = BPW // LANES
CT = 4096
NTILE = 16
CT3 = NUM_ACT * CT
CT_SLICE = CT3 // NTILE

_ct3 = np.zeros((CT3,), dtype=np.int64)
for _a in range(NUM_ACT):
    for _j in range(CT):
        _t = _j >> 7
        _c0 = (_j >> 4) & 7
        _c1 = (_j & 15) - 8
        _h = (_t * 2654435761 + _c0 * 1013904223 + _c1 * 1664525) % (1 << 32) % IHT
        _ct3[_a * CT + _j] = _a * IHT + _h
_CT3_IDX = _ct3.astype(np.int32)

_KT8 = [np.int32((_t << 7) + 8) for _t in range(T)]


def _ifloor(x):
    t = x.astype(jnp.int32)
    return jnp.where(t.astype(jnp.float32) > x, t - np.int32(1), t)


def _body(state_hbm, wflat_hbm, ct_hbm, out_hbm,
          ang_v, vel_v, ctl_v, gw_v, ctw_v, keys_v, accst, out_stage, shared, sem):
    cid = lax.axis_index("c")
    sid = lax.axis_index("s")
    wid = sid * NC + cid
    base = wid * BPW

    pltpu.sync_copy(ct_hbm.at[pl.ds(sid * CT_SLICE, CT_SLICE)], ctl_v)
    cp = pltpu.async_copy(wflat_hbm.at[ctl_v], gw_v, sem)
    pltpu.sync_copy(state_hbm.at[0, pl.ds(base, BPW)], ang_v)
    pltpu.sync_copy(state_hbm.at[1, pl.ds(base, BPW)], vel_v)
    cp.wait()
    pltpu.sync_copy(gw_v, shared.at[pl.ds(sid * CT_SLICE, CT_SLICE)])

    for c in range(CH):
        a = ang_v[pl.ds(c * LANES, LANES)]
        v = vel_v[pl.ds(c * LANES, LANES)]
        a_s = (a + np.float32(-np.pi)) / np.float32(np.pi * 2.0) * np.float32(8)
        v_s = (v + np.float32(-np.pi * 2.0)) / np.float32(4.0 * np.pi) * np.float32(8)
        q0 = _ifloor(a_s * np.float32(32))
        q1 = _ifloor(v_s * np.float32(32))
        for t in range(T):
            kp = jnp.bitwise_and((q0 + np.int32(t)) >> 1, np.int32(0x70))
            c1 = (q1 + np.int32((3 * t) & 31)) >> 5
            key = (_KT8[t] + kp) + c1
            keys_v[pl.ds(t * BPW + c * LANES, LANES)] = key

    plsc.subcore_barrier()

    pltpu.sync_copy(shared, ctw_v)

    lane = lax.iota(jnp.int32, LANES)
    tsel = (lane % np.int32(3)) * np.int32(LANES)
    for c in range(CH):
        acc0 = acc1 = acc2 = None
        for t in range(T):
            key = keys_v[pl.ds(t * BPW + c * LANES, LANES)]
            g0v = plsc.load_gather(ctw_v, [key])
            g1v = plsc.load_gather(ctw_v, [key + np.int32(CT)])
            g2v = plsc.load_gather(ctw_v, [key + np.int32(2 * CT)])
            acc0 = g0v if acc0 is None else acc0 + g0v
            acc1 = g1v if acc1 is None else acc1 + g1v
            acc2 = g2v if acc2 is None else acc2 + g2v
        accst[pl.ds(0, LANES)] = acc0
        accst[pl.ds(LANES, LANES)] = acc1
        accst[pl.ds(2 * LANES, LANES)] = acc2
        for i in range(LANES):
            row = plsc.load_gather(accst, [tsel + np.int32(i)])
            out_stage[pl.ds((c * LANES + i) * LANES, LANES)] = row

    pltpu.sync_copy(out_stage, out_hbm.at[pl.ds(base * LANES, BPW * LANES)])


@functools.cache
def _make_sc_call():
    mesh = plsc.VectorSubcoreMesh(core_axis_name="c", subcore_axis_name="s")
    return pl.kernel(
        _body,
        out_type=jax.ShapeDtypeStruct((B * LANES,), jnp.float32),
        mesh=mesh,
        compiler_params=pltpu.CompilerParams(use_tc_tiling_on_sc=False,
                                             needs_layout_passes=False),
        scratch_types=[
            pltpu.VMEM((BPW,), jnp.float32),
            pltpu.VMEM((BPW,), jnp.float32),
            pltpu.VMEM((CT_SLICE,), jnp.int32),
            pltpu.VMEM((CT_SLICE,), jnp.float32),
            pltpu.VMEM((CT3,), jnp.float32),
            pltpu.VMEM((T * BPW,), jnp.int32),
            pltpu.VMEM((NUM_ACT * LANES,), jnp.float32),
            pltpu.VMEM((BPW * LANES,), jnp.float32),
            pltpu.VMEM_SHARED((CT3,), jnp.float32),
            pltpu.SemaphoreType.DMA,
        ],
    )


def kernel(state, W, b):
    out = _make_sc_call()(state, W.reshape(NUM_ACT * IHT), jnp.asarray(_CT3_IDX))
    return out.reshape(B, LANES)[:, :NUM_ACT] + b


# speedup vs baseline: 1.0182x; 1.0182x over previous
import functools
import numpy as np
import jax
import jax.numpy as jnp
from jax import lax
from jax.experimental import pallas as pl
from jax.experimental.pallas import tpu as pltpu
from jax.experimental.pallas import tpu_sc as plsc

NUM_ACT = 3
IHT = 65536
T = 32
B = 2048
LANES = 16
NC = 2
NW = 32
BPW = B // NW
CH = BPW // LANES
CT = 4096
NTILE = 16
CT3 = NUM_ACT * CT
CT_SLICE = CT3 // NTILE

_ct3 = np.zeros((CT3,), dtype=np.int64)
for _a in range(NUM_ACT):
    for _j in range(CT):
        _t = _j >> 7
        _c0 = (_j >> 4) & 7
        _c1 = (_j & 15) - 8
        _h = (_t * 2654435761 + _c0 * 1013904223 + _c1 * 1664525) % (1 << 32) % IHT
        _ct3[_a * CT + _j] = _a * IHT + _h
_CT3_IDX = _ct3.astype(np.int32)

_KT8 = [np.int32((_t << 7) + 8) for _t in range(T)]


def _ifloor(x):
    t = x.astype(jnp.int32)
    return jnp.where(t.astype(jnp.float32) > x, t - np.int32(1), t)


def _body(state_hbm, wflat_hbm, ct_hbm, out_hbm,
          ang_v, vel_v, ctl_v, gw_v, ctw_v, accst, out_stage, shared, sem):
    cid = lax.axis_index("c")
    sid = lax.axis_index("s")
    wid = sid * NC + cid
    base = wid * BPW

    pltpu.sync_copy(ct_hbm.at[pl.ds(sid * CT_SLICE, CT_SLICE)], ctl_v)
    cp = pltpu.async_copy(wflat_hbm.at[ctl_v], gw_v, sem)
    pltpu.sync_copy(state_hbm.at[0, pl.ds(base, BPW)], ang_v)
    pltpu.sync_copy(state_hbm.at[1, pl.ds(base, BPW)], vel_v)
    cp.wait()
    pltpu.sync_copy(gw_v, shared.at[pl.ds(sid * CT_SLICE, CT_SLICE)])

    plsc.subcore_barrier()

    pltpu.sync_copy(shared, ctw_v)

    lane = lax.iota(jnp.int32, LANES)
    tsel = (lane % np.int32(3)) * np.int32(LANES)
    for c in range(CH):
        a = ang_v[pl.ds(c * LANES, LANES)]
        v = vel_v[pl.ds(c * LANES, LANES)]
        a_s = (a + np.float32(-np.pi)) / np.float32(np.pi * 2.0) * np.float32(8)
        v_s = (v + np.float32(-np.pi * 2.0)) / np.float32(4.0 * np.pi) * np.float32(8)
        q0 = _ifloor(a_s * np.float32(32))
        q1 = _ifloor(v_s * np.float32(32))
        acc0 = acc1 = acc2 = None
        for t in range(T):
            kp = jnp.bitwise_and((q0 + np.int32(t)) >> 1, np.int32(0x70))
            c1 = (q1 + np.int32((3 * t) & 31)) >> 5
            key = (_KT8[t] + kp) + c1
            g0v = plsc.load_gather(ctw_v, [key])
            g1v = plsc.load_gather(ctw_v, [key + np.int32(CT)])
            g2v = plsc.load_gather(ctw_v, [key + np.int32(2 * CT)])
            acc0 = g0v if acc0 is None else acc0 + g0v
            acc1 = g1v if acc1 is None else acc1 + g1v
            acc2 = g2v if acc2 is None else acc2 + g2v
        accst[pl.ds(0, LANES)] = acc0
        accst[pl.ds(LANES, LANES)] = acc1
        accst[pl.ds(2 * LANES, LANES)] = acc2
        for i in range(LANES):
            row = plsc.load_gather(accst, [tsel + np.int32(i)])
            out_stage[pl.ds((c * LANES + i) * LANES, LANES)] = row

    pltpu.sync_copy(out_stage, out_hbm.at[pl.ds(base * LANES, BPW * LANES)])


@functools.cache
def _make_sc_call():
    mesh = plsc.VectorSubcoreMesh(core_axis_name="c", subcore_axis_name="s")
    return pl.kernel(
        _body,
        out_type=jax.ShapeDtypeStruct((B * LANES,), jnp.float32),
        mesh=mesh,
        compiler_params=pltpu.CompilerParams(use_tc_tiling_on_sc=False,
                                             needs_layout_passes=False),
        scratch_types=[
            pltpu.VMEM((BPW,), jnp.float32),
            pltpu.VMEM((BPW,), jnp.float32),
            pltpu.VMEM((CT_SLICE,), jnp.int32),
            pltpu.VMEM((CT_SLICE,), jnp.float32),
            pltpu.VMEM((CT3,), jnp.float32),
            pltpu.VMEM((NUM_ACT * LANES,), jnp.float32),
            pltpu.VMEM((BPW * LANES,), jnp.float32),
            pltpu.VMEM_SHARED((CT3,), jnp.float32),
            pltpu.SemaphoreType.DMA,
        ],
    )


def kernel(state, W, b):
    out = _make_sc_call()(state, W.reshape(NUM_ACT * IHT), jnp.asarray(_CT3_IDX))
    return out.reshape(B, LANES)[:, :NUM_ACT] + b
